# Initial kernel scaffold; baseline (speedup 1.0000x reference)
import jax
import jax.numpy as jnp
from jax.experimental import pallas as pl


def kernel(x, pos, W, b):
    raise NotImplementedError("write your pallas kernel here")


# speedup vs baseline: 37.0953x; 37.0953x over previous
import functools

import jax
import jax.numpy as jnp
from jax.experimental import pallas as pl
from jax.experimental.pallas import tpu as pltpu

_K = 64
_R2 = 0.2 * 0.2
_CHUNK = 256
_STILE = 256
_NEG = -1e30


def _prep_body(posT_ref, xT_ref, pos_ref, W_ref, WT_ref, bT_ref, GT_ref, P_ref):
    posT = posT_ref[0]
    xT = xT_ref[0]
    pos = pos_ref[0]
    W = W_ref[...]
    WT = WT_ref[...]
    bT = bT_ref[...]

    cf = xT.shape[0]
    gt = jnp.broadcast_to(bT, (WT.shape[0], posT.shape[1])).astype(jnp.float32)
    for k in range(3):
        gt = gt + WT[:, k:k + 1] * posT[k:k + 1, :]
    for k in range(cf):
        gt = gt + WT[:, 3 + k:4 + k] * xT[k:k + 1, :]
    GT_ref[0] = gt

    p = pos[:, 0:1] * W[0:1, :]
    for k in range(1, 3):
        p = p + pos[:, k:k + 1] * W[k:k + 1, :]
    P_ref[0] = p


def _main_body(pos_tile_ref, posT_ref, GT_ref, P_ref, out_ref, sel_ref):
    ps = pos_tile_ref[0]
    pnT = posT_ref[0]
    n = pnT.shape[1]
    s = ps.shape[0]

    ps0, ps1, ps2 = ps[:, 0:1], ps[:, 1:2], ps[:, 2:3]
    pn0, pn1, pn2 = pnT[0:1, :], pnT[1:2, :], pnT[2:3, :]
    sq_s = ps0 * ps0 + ps1 * ps1 + ps2 * ps2
    sq_n = pn0 * pn0 + pn1 * pn1 + pn2 * pn2

    row = jax.lax.broadcasted_iota(jnp.int32, (_CHUNK, _CHUNK), 0)
    col = jax.lax.broadcasted_iota(jnp.int32, (_CHUNK, _CHUNK), 1)
    ut = (row < col).astype(jnp.bfloat16)

    carry = jnp.zeros((s, 1), dtype=jnp.float32)
    ps_bf = ps.astype(jnp.bfloat16)
    pnT_bf = pnT.astype(jnp.bfloat16)
    for j in range(n // _CHUNK):
        lo, hi = j * _CHUNK, (j + 1) * _CHUNK
        dot = jnp.dot(ps_bf, pnT_bf[:, lo:hi],
                      preferred_element_type=jnp.float32)
        d2 = (sq_s + sq_n[:, lo:hi]) - 2.0 * dot
        wn = d2 < _R2
        wf = wn.astype(jnp.float32)
        rank = jnp.dot(wf.astype(jnp.bfloat16), ut,
                       preferred_element_type=jnp.float32)
        sel = wn & ((carry + rank) < float(_K))
        sel_ref[:, lo:hi] = sel.astype(jnp.float32)
        carry = carry + jnp.sum(wf, axis=1, keepdims=True)

    gt = GT_ref[0]
    cols = []
    for c in range(gt.shape[0]):
        grow = gt[c:c + 1, :]
        masked = jnp.where(sel_ref[:, :] != 0.0, grow, _NEG)
        cols.append(jnp.max(masked, axis=1, keepdims=True))
    m = jnp.concatenate(cols, axis=1)
    out_ref[0] = jnp.maximum(m - P_ref[0], 0.0)


@functools.partial(jax.jit, static_argnames=())
def kernel(x, pos, W, b):
    B, N, Cf = x.shape
    Cout = W.shape[1]
    posT = jnp.transpose(pos, (0, 2, 1))
    xT = jnp.transpose(x, (0, 2, 1))
    WT = jnp.transpose(W)
    bT = b.reshape(Cout, 1)

    GT, P = pl.pallas_call(
        _prep_body,
        grid=(B,),
        in_specs=[
            pl.BlockSpec((1, 3, N), lambda bi: (bi, 0, 0)),
            pl.BlockSpec((1, Cf, N), lambda bi: (bi, 0, 0)),
            pl.BlockSpec((1, N, 3), lambda bi: (bi, 0, 0)),
            pl.BlockSpec((3 + Cf, Cout), lambda bi: (0, 0)),
            pl.BlockSpec((Cout, 3 + Cf), lambda bi: (0, 0)),
            pl.BlockSpec((Cout, 1), lambda bi: (0, 0)),
        ],
        out_specs=[
            pl.BlockSpec((1, Cout, N), lambda bi: (bi, 0, 0)),
            pl.BlockSpec((1, N, Cout), lambda bi: (bi, 0, 0)),
        ],
        out_shape=[
            jax.ShapeDtypeStruct((B, Cout, N), jnp.float32),
            jax.ShapeDtypeStruct((B, N, Cout), jnp.float32),
        ],
    )(posT, xT, pos, W, WT, bT)

    out = pl.pallas_call(
        _main_body,
        grid=(B, N // _STILE),
        in_specs=[
            pl.BlockSpec((1, _STILE, 3), lambda bi, si: (bi, si, 0)),
            pl.BlockSpec((1, 3, N), lambda bi, si: (bi, 0, 0)),
            pl.BlockSpec((1, Cout, N), lambda bi, si: (bi, 0, 0)),
            pl.BlockSpec((1, _STILE, Cout), lambda bi, si: (bi, si, 0)),
        ],
        out_specs=pl.BlockSpec((1, _STILE, Cout), lambda bi, si: (bi, si, 0)),
        out_shape=jax.ShapeDtypeStruct((B, N, Cout), jnp.float32),
        scratch_shapes=[pltpu.VMEM((_STILE, N), jnp.float32)],
    )(pos, posT, GT, P)

    return (out, pos)


# speedup vs baseline: 62.5206x; 1.6854x over previous
import functools

import jax
import jax.numpy as jnp
from jax.experimental import pallas as pl
from jax.experimental.pallas import tpu as pltpu

_K = 64
_R2 = 0.2 * 0.2
_CHUNK = 256
_STILE = 256
_NEG = -1e30


def _prep_body(posT_ref, xT_ref, pos_ref, W_ref, WT_ref, bT_ref, GT_ref, P_ref):
    posT = posT_ref[0]
    xT = xT_ref[0]
    pos = pos_ref[0]
    W = W_ref[...]
    WT = WT_ref[...]
    bT = bT_ref[...]

    cf = xT.shape[0]
    gt = jnp.broadcast_to(bT, (WT.shape[0], posT.shape[1])).astype(jnp.float32)
    for k in range(3):
        gt = gt + WT[:, k:k + 1] * posT[k:k + 1, :]
    for k in range(cf):
        gt = gt + WT[:, 3 + k:4 + k] * xT[k:k + 1, :]
    GT_ref[0] = gt.astype(jnp.bfloat16)

    p = pos[:, 0:1] * W[0:1, :]
    for k in range(1, 3):
        p = p + pos[:, k:k + 1] * W[k:k + 1, :]
    P_ref[0] = p


def _main_body(pos_tile_ref, posT_ref, GT_ref, P_ref, out_ref, sel_ref):
    ps = pos_tile_ref[0]
    pnT = posT_ref[0]
    n = pnT.shape[1]
    s = ps.shape[0]

    ps0, ps1, ps2 = ps[:, 0:1], ps[:, 1:2], ps[:, 2:3]
    pn0, pn1, pn2 = pnT[0:1, :], pnT[1:2, :], pnT[2:3, :]
    sq_s = ps0 * ps0 + ps1 * ps1 + ps2 * ps2
    sq_n = pn0 * pn0 + pn1 * pn1 + pn2 * pn2

    row = jax.lax.broadcasted_iota(jnp.int32, (_CHUNK, _CHUNK), 0)
    col = jax.lax.broadcasted_iota(jnp.int32, (_CHUNK, _CHUNK), 1)
    ut = (row < col).astype(jnp.bfloat16)

    carry = jnp.zeros((s, 1), dtype=jnp.float32)
    ps_bf = ps.astype(jnp.bfloat16)
    pnT_bf = pnT.astype(jnp.bfloat16)
    for j in range(n // _CHUNK):
        lo, hi = j * _CHUNK, (j + 1) * _CHUNK
        dot = jnp.dot(ps_bf, pnT_bf[:, lo:hi],
                      preferred_element_type=jnp.float32)
        d2 = (sq_s + sq_n[:, lo:hi]) - 2.0 * dot
        wn = d2 < _R2
        wf = wn.astype(jnp.float32)
        rank = jnp.dot(wf.astype(jnp.bfloat16), ut,
                       preferred_element_type=jnp.float32)
        sel = wn & ((carry + rank) < float(_K))
        sel_ref[:, lo:hi] = jnp.where(sel, 0.0, _NEG).astype(jnp.bfloat16)
        carry = carry + jnp.sum(wf, axis=1, keepdims=True)

    gt = GT_ref[0]
    selm = sel_ref[:, :]
    cols = []
    for c in range(gt.shape[0]):
        masked = selm + gt[c:c + 1, :]
        cols.append(jnp.max(masked, axis=1, keepdims=True))
    m = jnp.concatenate(cols, axis=1).astype(jnp.float32)
    out_ref[0] = jnp.maximum(m - P_ref[0], 0.0)


@functools.partial(jax.jit, static_argnames=())
def kernel(x, pos, W, b):
    B, N, Cf = x.shape
    Cout = W.shape[1]
    posT = jnp.transpose(pos, (0, 2, 1))
    xT = jnp.transpose(x, (0, 2, 1))
    WT = jnp.transpose(W)
    bT = b.reshape(Cout, 1)

    GT, P = pl.pallas_call(
        _prep_body,
        grid=(B,),
        in_specs=[
            pl.BlockSpec((1, 3, N), lambda bi: (bi, 0, 0)),
            pl.BlockSpec((1, Cf, N), lambda bi: (bi, 0, 0)),
            pl.BlockSpec((1, N, 3), lambda bi: (bi, 0, 0)),
            pl.BlockSpec((3 + Cf, Cout), lambda bi: (0, 0)),
            pl.BlockSpec((Cout, 3 + Cf), lambda bi: (0, 0)),
            pl.BlockSpec((Cout, 1), lambda bi: (0, 0)),
        ],
        out_specs=[
            pl.BlockSpec((1, Cout, N), lambda bi: (bi, 0, 0)),
            pl.BlockSpec((1, N, Cout), lambda bi: (bi, 0, 0)),
        ],
        out_shape=[
            jax.ShapeDtypeStruct((B, Cout, N), jnp.bfloat16),
            jax.ShapeDtypeStruct((B, N, Cout), jnp.float32),
        ],
    )(posT, xT, pos, W, WT, bT)

    out = pl.pallas_call(
        _main_body,
        grid=(B, N // _STILE),
        in_specs=[
            pl.BlockSpec((1, _STILE, 3), lambda bi, si: (bi, si, 0)),
            pl.BlockSpec((1, 3, N), lambda bi, si: (bi, 0, 0)),
            pl.BlockSpec((1, Cout, N), lambda bi, si: (bi, 0, 0)),
            pl.BlockSpec((1, _STILE, Cout), lambda bi, si: (bi, si, 0)),
        ],
        out_specs=pl.BlockSpec((1, _STILE, Cout), lambda bi, si: (bi, si, 0)),
        out_shape=jax.ShapeDtypeStruct((B, N, Cout), jnp.float32),
        scratch_shapes=[pltpu.VMEM((_STILE, N), jnp.bfloat16)],
    )(pos, posT, GT, P)

    return (out, pos)


# speedup vs baseline: 63.6417x; 1.0179x over previous
import functools

import jax
import jax.numpy as jnp
from jax.experimental import pallas as pl
from jax.experimental.pallas import tpu as pltpu

_K = 64
_R2 = 0.2 * 0.2
_CHUNK = 256
_STILE = 256
_NEG = -1e30


def _prep_body(posT_ref, xT_ref, pos_ref, W_ref, WT_ref, bT_ref, GT_ref, P_ref):
    posT = posT_ref[0]
    xT = xT_ref[0]
    pos = pos_ref[0]
    W = W_ref[...]
    WT = WT_ref[...]
    bT = bT_ref[...]

    cf = xT.shape[0]
    gt = jnp.broadcast_to(bT, (WT.shape[0], posT.shape[1])).astype(jnp.float32)
    for k in range(3):
        gt = gt + WT[:, k:k + 1] * posT[k:k + 1, :]
    for k in range(cf):
        gt = gt + WT[:, 3 + k:4 + k] * xT[k:k + 1, :]
    GT_ref[0] = gt.astype(jnp.bfloat16)

    p = pos[:, 0:1] * W[0:1, :]
    for k in range(1, 3):
        p = p + pos[:, k:k + 1] * W[k:k + 1, :]
    P_ref[0] = p


def _main_body(pos_tile_ref, posT_ref, GT_ref, P_ref, out_ref, sel_ref):
    ps = pos_tile_ref[0]
    pnT = posT_ref[0]
    n = pnT.shape[1]
    s = ps.shape[0]

    ps0, ps1, ps2 = ps[:, 0:1], ps[:, 1:2], ps[:, 2:3]
    pn0, pn1, pn2 = pnT[0:1, :], pnT[1:2, :], pnT[2:3, :]
    sq_s = ps0 * ps0 + ps1 * ps1 + ps2 * ps2
    sq_n = pn0 * pn0 + pn1 * pn1 + pn2 * pn2

    row = jax.lax.broadcasted_iota(jnp.int32, (_CHUNK, _CHUNK), 0)
    col = jax.lax.broadcasted_iota(jnp.int32, (_CHUNK, _CHUNK), 1)
    ut = (row < col).astype(jnp.bfloat16)

    carry = jnp.zeros((s, 1), dtype=jnp.float32)
    ps_bf = ps.astype(jnp.bfloat16)
    pnT_bf = pnT.astype(jnp.bfloat16)
    sq_comb = sq_s + sq_n
    one_bf = jnp.bfloat16(1.0)
    zero_bf = jnp.bfloat16(0.0)
    neg_bf = jnp.bfloat16(_NEG)
    for j in range(n // _CHUNK):
        lo, hi = j * _CHUNK, (j + 1) * _CHUNK
        dot = jnp.dot(ps_bf, pnT_bf[:, lo:hi],
                      preferred_element_type=jnp.float32)
        d2 = sq_comb[:, lo:hi] - 2.0 * dot
        wn = d2 < _R2
        wf = wn.astype(jnp.float32)
        wbf = wf.astype(jnp.bfloat16)
        rank = jnp.dot(wbf, ut, preferred_element_type=jnp.float32)
        thresh = float(_K) - carry
        sel = wn & (rank < thresh)
        sel_ref[:, lo:hi] = jnp.where(sel, 0.0, _NEG).astype(jnp.bfloat16)
        carry = carry + (rank[:, _CHUNK - 1:] + wf[:, _CHUNK - 1:])

    gt = GT_ref[0]
    selm = sel_ref[:, :]
    cols = []
    for c0 in range(0, gt.shape[0], 4):
        for c in range(c0, c0 + 4):
            masked = selm + gt[c:c + 1, :]
            cols.append(jnp.max(masked, axis=1, keepdims=True))
    m = jnp.concatenate(cols, axis=1).astype(jnp.float32)
    out_ref[0] = jnp.maximum(m - P_ref[0], 0.0)


@functools.partial(jax.jit, static_argnames=())
def kernel(x, pos, W, b):
    B, N, Cf = x.shape
    Cout = W.shape[1]
    posT = jnp.transpose(pos, (0, 2, 1))
    xT = jnp.transpose(x, (0, 2, 1))
    WT = jnp.transpose(W)
    bT = b.reshape(Cout, 1)

    GT, P = pl.pallas_call(
        _prep_body,
        grid=(B,),
        in_specs=[
            pl.BlockSpec((1, 3, N), lambda bi: (bi, 0, 0)),
            pl.BlockSpec((1, Cf, N), lambda bi: (bi, 0, 0)),
            pl.BlockSpec((1, N, 3), lambda bi: (bi, 0, 0)),
            pl.BlockSpec((3 + Cf, Cout), lambda bi: (0, 0)),
            pl.BlockSpec((Cout, 3 + Cf), lambda bi: (0, 0)),
            pl.BlockSpec((Cout, 1), lambda bi: (0, 0)),
        ],
        out_specs=[
            pl.BlockSpec((1, Cout, N), lambda bi: (bi, 0, 0)),
            pl.BlockSpec((1, N, Cout), lambda bi: (bi, 0, 0)),
        ],
        out_shape=[
            jax.ShapeDtypeStruct((B, Cout, N), jnp.bfloat16),
            jax.ShapeDtypeStruct((B, N, Cout), jnp.float32),
        ],
    )(posT, xT, pos, W, WT, bT)

    out = pl.pallas_call(
        _main_body,
        grid=(B, N // _STILE),
        in_specs=[
            pl.BlockSpec((1, _STILE, 3), lambda bi, si: (bi, si, 0)),
            pl.BlockSpec((1, 3, N), lambda bi, si: (bi, 0, 0)),
            pl.BlockSpec((1, Cout, N), lambda bi, si: (bi, 0, 0)),
            pl.BlockSpec((1, _STILE, Cout), lambda bi, si: (bi, si, 0)),
        ],
        out_specs=pl.BlockSpec((1, _STILE, Cout), lambda bi, si: (bi, si, 0)),
        out_shape=jax.ShapeDtypeStruct((B, N, Cout), jnp.float32),
        scratch_shapes=[pltpu.VMEM((_STILE, N), jnp.bfloat16)],
    )(pos, posT, GT, P)

    return (out, pos)
